# Initial kernel scaffold; baseline (speedup 1.0000x reference)
import jax
import jax.numpy as jnp
from jax.experimental import pallas as pl


def kernel(x, edge_index, W1, b1, W2, b2):
    raise NotImplementedError("write your pallas kernel here")


# speedup vs baseline: 7.8126x; 7.8126x over previous
import functools

import jax
import jax.numpy as jnp
from jax import lax
from jax.experimental import pallas as pl
from jax.experimental.pallas import tpu as pltpu
from jax.experimental.pallas import tpu_sc as plsc

N = 10000
D = 128
E = 320000
N_PAD = 10240
NW = 32
EPW = 10240
E_PAD = NW * EPW
CHUNK = 128
ROWS_PER_SC = N_PAD // 16

_mesh = plsc.VectorSubcoreMesh(core_axis_name="c", subcore_axis_name="s")
_sc_params = pltpu.CompilerParams(needs_layout_passes=False)


@functools.partial(
    pl.kernel,
    out_type=jax.ShapeDtypeStruct((2, N_PAD), jnp.float32),
    mesh=_mesh,
    compiler_params=_sc_params,
    scratch_types=[
        pltpu.VMEM((CHUNK,), jnp.int32),
        pltpu.VMEM((N_PAD,), jnp.float32),
        pltpu.VMEM((ROWS_PER_SC,), jnp.float32),
        pltpu.VMEM((ROWS_PER_SC,), jnp.float32),
        pltpu.VMEM_SHARED((16, N_PAD), jnp.float32),
    ],
)
def _deg_kernel(dst_hbm, out_hbm, idx_v, hist_v, col_v, buf_v, slab):
    cid = lax.axis_index("c")
    sid = lax.axis_index("s")
    wid = sid * 2 + cid
    zero16 = jnp.zeros((16,), jnp.float32)
    ones16 = jnp.ones((16,), jnp.float32)

    def zero_hist(i, carry):
        hist_v[pl.ds(i * 16, 16)] = zero16
        return carry

    lax.fori_loop(0, N_PAD // 16, zero_hist, 0)

    def chunk_body(ci, carry):
        base = wid * EPW + ci * CHUNK
        pltpu.sync_copy(dst_hbm.at[pl.ds(base, CHUNK)], idx_v)

        def inner(j, c2):
            idx = idx_v[pl.ds(j * 16, 16)]
            plsc.addupdate_scatter(hist_v, [idx], ones16)
            return c2

        lax.fori_loop(0, CHUNK // 16, inner, 0)
        return carry

    lax.fori_loop(0, EPW // CHUNK, chunk_body, 0)

    pltpu.sync_copy(hist_v, slab.at[sid])
    plsc.subcore_barrier()

    col0 = sid * ROWS_PER_SC

    def zero_col(i, carry):
        col_v[pl.ds(i * 16, 16)] = zero16
        return carry

    lax.fori_loop(0, ROWS_PER_SC // 16, zero_col, 0)

    def row_body(r, carry):
        pltpu.sync_copy(slab.at[r, pl.ds(col0, ROWS_PER_SC)], buf_v)

        def addv(i, c2):
            col_v[pl.ds(i * 16, 16)] = (
                col_v[pl.ds(i * 16, 16)] + buf_v[pl.ds(i * 16, 16)]
            )
            return c2

        lax.fori_loop(0, ROWS_PER_SC // 16, addv, 0)
        return carry

    lax.fori_loop(0, 16, row_body, 0)
    pltpu.sync_copy(col_v, out_hbm.at[cid, pl.ds(col0, ROWS_PER_SC)])


@functools.partial(
    pl.kernel,
    out_type=jax.ShapeDtypeStruct((2, N_PAD, D), jnp.float32),
    mesh=_mesh,
    compiler_params=_sc_params,
    scratch_types=[
        pltpu.VMEM((CHUNK,), jnp.int32),
        pltpu.VMEM((CHUNK,), jnp.int32),
        pltpu.VMEM((CHUNK, D), jnp.float32),
        pltpu.VMEM_SHARED((N_PAD, D), jnp.float32),
        pltpu.SemaphoreType.DMA,
    ],
)
def _agg_kernel(h_hbm, src_hbm, dst_hbm, out_hbm, sidx_v, didx_v, rows_v,
                acc_sh, sem):
    cid = lax.axis_index("c")
    sid = lax.axis_index("s")
    wid = sid * 2 + cid
    r0 = sid * ROWS_PER_SC

    pltpu.sync_copy(h_hbm.at[pl.ds(r0, ROWS_PER_SC)],
                    acc_sh.at[pl.ds(r0, ROWS_PER_SC)])
    plsc.subcore_barrier()

    def chunk_body(ci, carry):
        base = wid * EPW + ci * CHUNK
        pltpu.sync_copy(src_hbm.at[pl.ds(base, CHUNK)], sidx_v)
        pltpu.sync_copy(dst_hbm.at[pl.ds(base, CHUNK)], didx_v)
        pltpu.async_copy(h_hbm.at[sidx_v], rows_v, sem).wait()
        pltpu.sync_copy(rows_v, acc_sh.at[didx_v], add=True)
        return carry

    lax.fori_loop(0, EPW // CHUNK, chunk_body, 0)
    plsc.subcore_barrier()
    pltpu.sync_copy(acc_sh.at[pl.ds(r0, ROWS_PER_SC)],
                    out_hbm.at[cid, pl.ds(r0, ROWS_PER_SC)])


_BLK = 1024
_GRID = N_PAD // _BLK


def _h_body(p_ref, x_ref, w_ref, o_ref):
    deg = 1.0 + p_ref[0] + p_ref[1]
    dinv = lax.rsqrt(deg)
    h = lax.dot_general(x_ref[...], w_ref[...], (((1,), (1,)), ((), ())),
                        preferred_element_type=jnp.float32,
                        precision=lax.Precision.HIGHEST)
    o_ref[...] = dinv * h


def _first_layer(p, x_pad, W1):
    return pl.pallas_call(
        _h_body,
        grid=(_GRID,),
        in_specs=[
            pl.BlockSpec((2, _BLK, 1), lambda i: (0, i, 0)),
            pl.BlockSpec((_BLK, D), lambda i: (i, 0)),
            pl.BlockSpec((D, D), lambda i: (0, 0)),
        ],
        out_specs=pl.BlockSpec((_BLK, D), lambda i: (i, 0)),
        out_shape=jax.ShapeDtypeStruct((N_PAD, D), jnp.float32),
    )(p, x_pad, W1)


def _mid_body(p_ref, acc_ref, h_ref, b_ref, w_ref, o_ref):
    deg = 1.0 + p_ref[0] + p_ref[1]
    dinv = lax.rsqrt(deg)
    agg = acc_ref[0] + acc_ref[1] - h_ref[...]
    x2 = dinv * agg + b_ref[...]
    h2 = lax.dot_general(x2, w_ref[...], (((1,), (1,)), ((), ())),
                         preferred_element_type=jnp.float32,
                         precision=lax.Precision.HIGHEST)
    o_ref[...] = dinv * h2


def _mid_layer(p, acc, h, b1, W2):
    return pl.pallas_call(
        _mid_body,
        grid=(_GRID,),
        in_specs=[
            pl.BlockSpec((2, _BLK, 1), lambda i: (0, i, 0)),
            pl.BlockSpec((2, _BLK, D), lambda i: (0, i, 0)),
            pl.BlockSpec((_BLK, D), lambda i: (i, 0)),
            pl.BlockSpec((1, D), lambda i: (0, 0)),
            pl.BlockSpec((D, D), lambda i: (0, 0)),
        ],
        out_specs=pl.BlockSpec((_BLK, D), lambda i: (i, 0)),
        out_shape=jax.ShapeDtypeStruct((N_PAD, D), jnp.float32),
    )(p, acc, h, b1, W2)


def _final_body(p_ref, acc_ref, h_ref, b_ref, o_ref):
    deg = 1.0 + p_ref[0] + p_ref[1]
    dinv = lax.rsqrt(deg)
    agg = acc_ref[0] + acc_ref[1] - h_ref[...]
    o_ref[...] = dinv * agg + b_ref[...]


def _final_layer(p, acc, h, b2):
    return pl.pallas_call(
        _final_body,
        grid=(_GRID,),
        in_specs=[
            pl.BlockSpec((2, _BLK, 1), lambda i: (0, i, 0)),
            pl.BlockSpec((2, _BLK, D), lambda i: (0, i, 0)),
            pl.BlockSpec((_BLK, D), lambda i: (i, 0)),
            pl.BlockSpec((1, D), lambda i: (0, 0)),
        ],
        out_specs=pl.BlockSpec((_BLK, D), lambda i: (i, 0)),
        out_shape=jax.ShapeDtypeStruct((N_PAD, D), jnp.float32),
    )(p, acc, h, b2)


@jax.jit
def kernel(x, edge_index, W1, b1, W2, b2):
    src = edge_index[0].astype(jnp.int32)
    dst = edge_index[1].astype(jnp.int32)
    pad = jnp.full((E_PAD - E,), N, jnp.int32)
    src_pad = jnp.concatenate([src, pad])
    dst_pad = jnp.concatenate([dst, pad])
    x_pad = jnp.pad(x, ((0, N_PAD - N), (0, 0)))
    b1r = b1.reshape(1, D)
    b2r = b2.reshape(1, D)

    p = _deg_kernel(dst_pad)
    pcol = p.reshape(2, N_PAD, 1)

    h1 = _first_layer(pcol, x_pad, W1)
    acc1 = _agg_kernel(h1, src_pad, dst_pad)
    h2 = _mid_layer(pcol, acc1, h1, b1r, W2)
    acc2 = _agg_kernel(h2, src_pad, dst_pad)
    out = _final_layer(pcol, acc2, h2, b2r)
    return out[:N]


# speedup vs baseline: 9.7283x; 1.2452x over previous
import functools

import jax
import jax.numpy as jnp
from jax import lax
from jax.experimental import pallas as pl
from jax.experimental.pallas import tpu as pltpu
from jax.experimental.pallas import tpu_sc as plsc

N = 10000
D = 128
E = 320000
N_PAD = 10240
NW = 32
EPW = 10240
E_PAD = NW * EPW
CHUNK = 128
ROWS_PER_SC = N_PAD // 16

_mesh = plsc.VectorSubcoreMesh(core_axis_name="c", subcore_axis_name="s")
_sc_params = pltpu.CompilerParams(needs_layout_passes=False)


@functools.partial(
    pl.kernel,
    out_type=jax.ShapeDtypeStruct((2, N_PAD), jnp.float32),
    mesh=_mesh,
    compiler_params=_sc_params,
    scratch_types=[
        pltpu.VMEM((CHUNK,), jnp.int32),
        pltpu.VMEM((N_PAD,), jnp.float32),
        pltpu.VMEM((ROWS_PER_SC,), jnp.float32),
        pltpu.VMEM((ROWS_PER_SC,), jnp.float32),
        pltpu.VMEM_SHARED((16, N_PAD), jnp.float32),
    ],
)
def _deg_kernel(dst_hbm, out_hbm, idx_v, hist_v, col_v, buf_v, slab):
    cid = lax.axis_index("c")
    sid = lax.axis_index("s")
    wid = sid * 2 + cid
    zero16 = jnp.zeros((16,), jnp.float32)
    ones16 = jnp.ones((16,), jnp.float32)

    def zero_hist(i, carry):
        hist_v[pl.ds(i * 16, 16)] = zero16
        return carry

    lax.fori_loop(0, N_PAD // 16, zero_hist, 0)

    def chunk_body(ci, carry):
        base = wid * EPW + ci * CHUNK
        pltpu.sync_copy(dst_hbm.at[pl.ds(base, CHUNK)], idx_v)

        def inner(j, c2):
            idx = idx_v[pl.ds(j * 16, 16)]
            plsc.addupdate_scatter(hist_v, [idx], ones16)
            return c2

        lax.fori_loop(0, CHUNK // 16, inner, 0)
        return carry

    lax.fori_loop(0, EPW // CHUNK, chunk_body, 0)

    pltpu.sync_copy(hist_v, slab.at[sid])
    plsc.subcore_barrier()

    col0 = sid * ROWS_PER_SC

    def zero_col(i, carry):
        col_v[pl.ds(i * 16, 16)] = zero16
        return carry

    lax.fori_loop(0, ROWS_PER_SC // 16, zero_col, 0)

    def row_body(r, carry):
        pltpu.sync_copy(slab.at[r, pl.ds(col0, ROWS_PER_SC)], buf_v)

        def addv(i, c2):
            col_v[pl.ds(i * 16, 16)] = (
                col_v[pl.ds(i * 16, 16)] + buf_v[pl.ds(i * 16, 16)]
            )
            return c2

        lax.fori_loop(0, ROWS_PER_SC // 16, addv, 0)
        return carry

    lax.fori_loop(0, 16, row_body, 0)
    pltpu.sync_copy(col_v, out_hbm.at[cid, pl.ds(col0, ROWS_PER_SC)])


NCHUNK = EPW // CHUNK
NBUF = 2
NGROUP = NCHUNK // NBUF


@functools.partial(
    pl.kernel,
    out_type=jax.ShapeDtypeStruct((2, N_PAD, D), jnp.float32),
    mesh=_mesh,
    compiler_params=_sc_params,
    scratch_types=(
        [pltpu.VMEM((2, NBUF, CHUNK), jnp.int32),
         pltpu.VMEM((2, NBUF, CHUNK), jnp.int32)]
        + [pltpu.VMEM((CHUNK, D), jnp.float32) for _ in range(NBUF)]
        + [pltpu.VMEM_SHARED((N_PAD, D), jnp.float32)]
        + [pltpu.SemaphoreType.DMA for _ in range(2 + 2 * NBUF)]
    ),
)
def _agg_kernel(h_hbm, src_hbm, dst_hbm, out_hbm, sidx_v, didx_v, *rest):
    rows = rest[:NBUF]
    acc_sh = rest[NBUF]
    isem = rest[NBUF + 1:NBUF + 3]
    gsem = rest[NBUF + 3:NBUF + 3 + NBUF]
    ssem = rest[NBUF + 3 + NBUF:]
    cid = lax.axis_index("c")
    sid = lax.axis_index("s")
    wid = sid * 2 + cid
    r0 = sid * ROWS_PER_SC
    row0 = wid * NCHUNK

    pltpu.sync_copy(h_hbm.at[pl.ds(r0, ROWS_PER_SC)],
                    acc_sh.at[pl.ds(r0, ROWS_PER_SC)])
    plsc.subcore_barrier()

    def load_idx(g, p):
        blk = pl.ds(row0 + g * NBUF, NBUF)
        pltpu.async_copy(src_hbm.at[blk], sidx_v.at[p], isem[p])
        pltpu.async_copy(dst_hbm.at[blk], didx_v.at[p], isem[p])

    def wait_idx(p):
        pltpu.make_async_copy(src_hbm.at[pl.ds(0, NBUF)], sidx_v.at[p],
                              isem[p]).wait()
        pltpu.make_async_copy(dst_hbm.at[pl.ds(0, NBUF)], didx_v.at[p],
                              isem[p]).wait()

    load_idx(0, 0)
    load_idx(1, 1)
    wait_idx(0)
    for b in range(NBUF):
        pltpu.async_copy(h_hbm.at[sidx_v.at[0, b]], rows[b], gsem[b])

    def half_body(g, p):
        pnext = 1 - p
        for b in range(NBUF):
            pltpu.make_async_copy(h_hbm.at[sidx_v.at[p, b]], rows[b],
                                  gsem[b]).wait()
            pltpu.async_copy(rows[b], acc_sh.at[didx_v.at[p, b]], ssem[b],
                             add=True)
        @pl.when(g + 1 < NGROUP)
        def _():
            wait_idx(pnext)
            for b in range(NBUF):
                pltpu.make_async_copy(rows[b], acc_sh.at[didx_v.at[p, b]],
                                      ssem[b]).wait()
                pltpu.async_copy(h_hbm.at[sidx_v.at[pnext, b]], rows[b],
                                 gsem[b])

        @pl.when(g + 2 < NGROUP)
        def _():
            load_idx(g + 2, p)

    def group_body(q, carry):
        half_body(2 * q, 0)
        half_body(2 * q + 1, 1)
        return carry

    lax.fori_loop(0, NGROUP // 2, group_body, 0)
    for b in range(NBUF):
        pltpu.make_async_copy(rows[b], acc_sh.at[didx_v.at[(NGROUP - 1) % 2,
                                                           b]],
                              ssem[b]).wait()

    plsc.subcore_barrier()
    pltpu.sync_copy(acc_sh.at[pl.ds(r0, ROWS_PER_SC)],
                    out_hbm.at[cid, pl.ds(r0, ROWS_PER_SC)])


_BLK = 1024
_GRID = N_PAD // _BLK


def _h_body(p_ref, x_ref, w_ref, o_ref):
    deg = 1.0 + p_ref[0] + p_ref[1]
    dinv = lax.rsqrt(deg)
    h = lax.dot_general(x_ref[...], w_ref[...], (((1,), (1,)), ((), ())),
                        preferred_element_type=jnp.float32,
                        precision=lax.Precision.HIGHEST)
    o_ref[...] = dinv * h


def _first_layer(p, x_pad, W1):
    return pl.pallas_call(
        _h_body,
        grid=(_GRID,),
        in_specs=[
            pl.BlockSpec((2, _BLK, 1), lambda i: (0, i, 0)),
            pl.BlockSpec((_BLK, D), lambda i: (i, 0)),
            pl.BlockSpec((D, D), lambda i: (0, 0)),
        ],
        out_specs=pl.BlockSpec((_BLK, D), lambda i: (i, 0)),
        out_shape=jax.ShapeDtypeStruct((N_PAD, D), jnp.float32),
    )(p, x_pad, W1)


def _mid_body(p_ref, acc_ref, h_ref, b_ref, w_ref, o_ref):
    deg = 1.0 + p_ref[0] + p_ref[1]
    dinv = lax.rsqrt(deg)
    agg = acc_ref[0] + acc_ref[1] - h_ref[...]
    x2 = dinv * agg + b_ref[...]
    h2 = lax.dot_general(x2, w_ref[...], (((1,), (1,)), ((), ())),
                         preferred_element_type=jnp.float32,
                         precision=lax.Precision.HIGHEST)
    o_ref[...] = dinv * h2


def _mid_layer(p, acc, h, b1, W2):
    return pl.pallas_call(
        _mid_body,
        grid=(_GRID,),
        in_specs=[
            pl.BlockSpec((2, _BLK, 1), lambda i: (0, i, 0)),
            pl.BlockSpec((2, _BLK, D), lambda i: (0, i, 0)),
            pl.BlockSpec((_BLK, D), lambda i: (i, 0)),
            pl.BlockSpec((1, D), lambda i: (0, 0)),
            pl.BlockSpec((D, D), lambda i: (0, 0)),
        ],
        out_specs=pl.BlockSpec((_BLK, D), lambda i: (i, 0)),
        out_shape=jax.ShapeDtypeStruct((N_PAD, D), jnp.float32),
    )(p, acc, h, b1, W2)


def _final_body(p_ref, acc_ref, h_ref, b_ref, o_ref):
    deg = 1.0 + p_ref[0] + p_ref[1]
    dinv = lax.rsqrt(deg)
    agg = acc_ref[0] + acc_ref[1] - h_ref[...]
    o_ref[...] = dinv * agg + b_ref[...]


def _final_layer(p, acc, h, b2):
    return pl.pallas_call(
        _final_body,
        grid=(_GRID,),
        in_specs=[
            pl.BlockSpec((2, _BLK, 1), lambda i: (0, i, 0)),
            pl.BlockSpec((2, _BLK, D), lambda i: (0, i, 0)),
            pl.BlockSpec((_BLK, D), lambda i: (i, 0)),
            pl.BlockSpec((1, D), lambda i: (0, 0)),
        ],
        out_specs=pl.BlockSpec((_BLK, D), lambda i: (i, 0)),
        out_shape=jax.ShapeDtypeStruct((N_PAD, D), jnp.float32),
    )(p, acc, h, b2)


@jax.jit
def kernel(x, edge_index, W1, b1, W2, b2):
    src = edge_index[0].astype(jnp.int32)
    dst = edge_index[1].astype(jnp.int32)
    pad = jnp.full((E_PAD - E,), N, jnp.int32)
    src_pad = jnp.concatenate([src, pad])
    dst_pad = jnp.concatenate([dst, pad])
    src_m = src_pad.reshape(NW * NCHUNK, CHUNK)
    dst_m = dst_pad.reshape(NW * NCHUNK, CHUNK)
    x_pad = jnp.pad(x, ((0, N_PAD - N), (0, 0)))
    b1r = b1.reshape(1, D)
    b2r = b2.reshape(1, D)

    p = _deg_kernel(dst_pad)
    pcol = p.reshape(2, N_PAD, 1)

    h1 = _first_layer(pcol, x_pad, W1)
    acc1 = _agg_kernel(h1, src_m, dst_m)
    h2 = _mid_layer(pcol, acc1, h1, b1r, W2)
    acc2 = _agg_kernel(h2, src_m, dst_m)
    out = _final_layer(pcol, acc2, h2, b2r)
    return out[:N]


# speedup vs baseline: 10.4506x; 1.0743x over previous
import functools

import jax
import jax.numpy as jnp
from jax import lax
from jax.experimental import pallas as pl
from jax.experimental.pallas import tpu as pltpu
from jax.experimental.pallas import tpu_sc as plsc

N = 10000
D = 128
E = 320000
N_PAD = 10240
NW = 32
EPW = 10240
E_PAD = NW * EPW
CHUNK = 128
ROWS_PER_SC = N_PAD // 16

_mesh = plsc.VectorSubcoreMesh(core_axis_name="c", subcore_axis_name="s")
_sc_params = pltpu.CompilerParams(needs_layout_passes=False)


@functools.partial(
    pl.kernel,
    out_type=jax.ShapeDtypeStruct((2, N_PAD), jnp.float32),
    mesh=_mesh,
    compiler_params=_sc_params,
    scratch_types=[
        pltpu.VMEM((CHUNK,), jnp.int32),
        pltpu.VMEM((N_PAD,), jnp.float32),
        pltpu.VMEM((ROWS_PER_SC,), jnp.float32),
        pltpu.VMEM((ROWS_PER_SC,), jnp.float32),
        pltpu.VMEM_SHARED((16, N_PAD), jnp.float32),
    ],
)
def _deg_kernel(dst_hbm, out_hbm, idx_v, hist_v, col_v, buf_v, slab):
    cid = lax.axis_index("c")
    sid = lax.axis_index("s")
    wid = sid * 2 + cid
    zero16 = jnp.zeros((16,), jnp.float32)
    ones16 = jnp.ones((16,), jnp.float32)

    def zero_hist(i, carry):
        hist_v[pl.ds(i * 16, 16)] = zero16
        return carry

    lax.fori_loop(0, N_PAD // 16, zero_hist, 0)

    def chunk_body(ci, carry):
        base = wid * EPW + ci * CHUNK
        pltpu.sync_copy(dst_hbm.at[pl.ds(base, CHUNK)], idx_v)

        def inner(j, c2):
            idx = idx_v[pl.ds(j * 16, 16)]
            plsc.addupdate_scatter(hist_v, [idx], ones16)
            return c2

        lax.fori_loop(0, CHUNK // 16, inner, 0)
        return carry

    lax.fori_loop(0, EPW // CHUNK, chunk_body, 0)

    pltpu.sync_copy(hist_v, slab.at[sid])
    plsc.subcore_barrier()

    col0 = sid * ROWS_PER_SC

    def zero_col(i, carry):
        col_v[pl.ds(i * 16, 16)] = zero16
        return carry

    lax.fori_loop(0, ROWS_PER_SC // 16, zero_col, 0)

    def row_body(r, carry):
        pltpu.sync_copy(slab.at[r, pl.ds(col0, ROWS_PER_SC)], buf_v)

        def addv(i, c2):
            col_v[pl.ds(i * 16, 16)] = (
                col_v[pl.ds(i * 16, 16)] + buf_v[pl.ds(i * 16, 16)]
            )
            return c2

        lax.fori_loop(0, ROWS_PER_SC // 16, addv, 0)
        return carry

    lax.fori_loop(0, 16, row_body, 0)
    pltpu.sync_copy(col_v, out_hbm.at[cid, pl.ds(col0, ROWS_PER_SC)])


NCHUNK_TOT = E_PAD // CHUNK
NBUF = 2
C0 = 120
C1 = 40
assert 16 * (C0 + C1) == NCHUNK_TOT
assert C0 % (2 * NBUF) == 0 and C1 % (2 * NBUF) == 0


@functools.partial(
    pl.kernel,
    out_type=jax.ShapeDtypeStruct((2, N_PAD, D), jnp.float32),
    mesh=_mesh,
    compiler_params=_sc_params,
    scratch_types=(
        [pltpu.VMEM((2, NBUF, CHUNK), jnp.int32),
         pltpu.VMEM((2, NBUF, CHUNK), jnp.int32)]
        + [pltpu.VMEM((CHUNK, D), jnp.float32) for _ in range(NBUF)]
        + [pltpu.VMEM_SHARED((N_PAD, D), jnp.float32)]
        + [pltpu.SemaphoreType.DMA for _ in range(2 + 2 * NBUF)]
    ),
)
def _agg_kernel(h_hbm, src_hbm, dst_hbm, out_hbm, sidx_v, didx_v, *rest):
    rows = rest[:NBUF]
    acc_sh = rest[NBUF]
    isem = rest[NBUF + 1:NBUF + 3]
    gsem = rest[NBUF + 3:NBUF + 3 + NBUF]
    ssem = rest[NBUF + 3 + NBUF:]
    cid = lax.axis_index("c")
    sid = lax.axis_index("s")
    r0 = sid * ROWS_PER_SC
    row0 = jnp.where(cid == 0, sid * C0, 16 * C0 + sid * C1)
    ngroup = jnp.where(cid == 0, C0 // NBUF, C1 // NBUF)

    pltpu.sync_copy(h_hbm.at[pl.ds(r0, ROWS_PER_SC)],
                    acc_sh.at[pl.ds(r0, ROWS_PER_SC)])
    plsc.subcore_barrier()

    def load_idx(g, p):
        blk = pl.ds(row0 + g * NBUF, NBUF)
        pltpu.async_copy(src_hbm.at[blk], sidx_v.at[p], isem[p])
        pltpu.async_copy(dst_hbm.at[blk], didx_v.at[p], isem[p])

    def wait_idx(p):
        pltpu.make_async_copy(src_hbm.at[pl.ds(0, NBUF)], sidx_v.at[p],
                              isem[p]).wait()
        pltpu.make_async_copy(dst_hbm.at[pl.ds(0, NBUF)], didx_v.at[p],
                              isem[p]).wait()

    load_idx(0, 0)
    load_idx(1, 1)
    wait_idx(0)
    for b in range(NBUF):
        pltpu.async_copy(h_hbm.at[sidx_v.at[0, b]], rows[b], gsem[b])

    def half_body(g, p):
        pnext = 1 - p
        for b in range(NBUF):
            pltpu.make_async_copy(h_hbm.at[sidx_v.at[p, b]], rows[b],
                                  gsem[b]).wait()
            pltpu.async_copy(rows[b], acc_sh.at[didx_v.at[p, b]], ssem[b],
                             add=True)
        @pl.when(g + 1 < ngroup)
        def _():
            wait_idx(pnext)
            for b in range(NBUF):
                pltpu.make_async_copy(rows[b], acc_sh.at[didx_v.at[p, b]],
                                      ssem[b]).wait()
                pltpu.async_copy(h_hbm.at[sidx_v.at[pnext, b]], rows[b],
                                 gsem[b])

        @pl.when(g + 2 < ngroup)
        def _():
            load_idx(g + 2, p)

    def group_body(q, carry):
        half_body(2 * q, 0)
        half_body(2 * q + 1, 1)
        return carry

    lax.fori_loop(0, ngroup // 2, group_body, 0)
    for b in range(NBUF):
        pltpu.make_async_copy(rows[b], acc_sh.at[didx_v.at[0, b]],
                              ssem[b]).wait()

    plsc.subcore_barrier()
    pltpu.sync_copy(acc_sh.at[pl.ds(r0, ROWS_PER_SC)],
                    out_hbm.at[cid, pl.ds(r0, ROWS_PER_SC)])


_BLK = 1024
_GRID = N_PAD // _BLK


def _h_body(p_ref, x_ref, w_ref, o_ref):
    deg = 1.0 + p_ref[0] + p_ref[1]
    dinv = lax.rsqrt(deg)
    h = lax.dot_general(x_ref[...], w_ref[...], (((1,), (1,)), ((), ())),
                        preferred_element_type=jnp.float32,
                        precision=lax.Precision.HIGHEST)
    o_ref[...] = dinv * h


def _first_layer(p, x_pad, W1):
    return pl.pallas_call(
        _h_body,
        grid=(_GRID,),
        in_specs=[
            pl.BlockSpec((2, _BLK, 1), lambda i: (0, i, 0)),
            pl.BlockSpec((_BLK, D), lambda i: (i, 0)),
            pl.BlockSpec((D, D), lambda i: (0, 0)),
        ],
        out_specs=pl.BlockSpec((_BLK, D), lambda i: (i, 0)),
        out_shape=jax.ShapeDtypeStruct((N_PAD, D), jnp.float32),
    )(p, x_pad, W1)


def _mid_body(p_ref, acc_ref, h_ref, b_ref, w_ref, o_ref):
    deg = 1.0 + p_ref[0] + p_ref[1]
    dinv = lax.rsqrt(deg)
    agg = acc_ref[0] + acc_ref[1] - h_ref[...]
    x2 = dinv * agg + b_ref[...]
    h2 = lax.dot_general(x2, w_ref[...], (((1,), (1,)), ((), ())),
                         preferred_element_type=jnp.float32,
                         precision=lax.Precision.HIGHEST)
    o_ref[...] = dinv * h2


def _mid_layer(p, acc, h, b1, W2):
    return pl.pallas_call(
        _mid_body,
        grid=(_GRID,),
        in_specs=[
            pl.BlockSpec((2, _BLK, 1), lambda i: (0, i, 0)),
            pl.BlockSpec((2, _BLK, D), lambda i: (0, i, 0)),
            pl.BlockSpec((_BLK, D), lambda i: (i, 0)),
            pl.BlockSpec((1, D), lambda i: (0, 0)),
            pl.BlockSpec((D, D), lambda i: (0, 0)),
        ],
        out_specs=pl.BlockSpec((_BLK, D), lambda i: (i, 0)),
        out_shape=jax.ShapeDtypeStruct((N_PAD, D), jnp.float32),
    )(p, acc, h, b1, W2)


def _final_body(p_ref, acc_ref, h_ref, b_ref, o_ref):
    deg = 1.0 + p_ref[0] + p_ref[1]
    dinv = lax.rsqrt(deg)
    agg = acc_ref[0] + acc_ref[1] - h_ref[...]
    o_ref[...] = dinv * agg + b_ref[...]


def _final_layer(p, acc, h, b2):
    return pl.pallas_call(
        _final_body,
        grid=(_GRID,),
        in_specs=[
            pl.BlockSpec((2, _BLK, 1), lambda i: (0, i, 0)),
            pl.BlockSpec((2, _BLK, D), lambda i: (0, i, 0)),
            pl.BlockSpec((_BLK, D), lambda i: (i, 0)),
            pl.BlockSpec((1, D), lambda i: (0, 0)),
        ],
        out_specs=pl.BlockSpec((_BLK, D), lambda i: (i, 0)),
        out_shape=jax.ShapeDtypeStruct((N_PAD, D), jnp.float32),
    )(p, acc, h, b2)


@jax.jit
def kernel(x, edge_index, W1, b1, W2, b2):
    src = edge_index[0].astype(jnp.int32)
    dst = edge_index[1].astype(jnp.int32)
    pad = jnp.full((E_PAD - E,), N, jnp.int32)
    src_pad = jnp.concatenate([src, pad])
    dst_pad = jnp.concatenate([dst, pad])
    src_m = src_pad.reshape(NCHUNK_TOT, CHUNK)
    dst_m = dst_pad.reshape(NCHUNK_TOT, CHUNK)
    x_pad = jnp.pad(x, ((0, N_PAD - N), (0, 0)))
    b1r = b1.reshape(1, D)
    b2r = b2.reshape(1, D)

    p = _deg_kernel(dst_pad)
    pcol = p.reshape(2, N_PAD, 1)

    h1 = _first_layer(pcol, x_pad, W1)
    acc1 = _agg_kernel(h1, src_m, dst_m)
    h2 = _mid_layer(pcol, acc1, h1, b1r, W2)
    acc2 = _agg_kernel(h2, src_m, dst_m)
    out = _final_layer(pcol, acc2, h2, b2r)
    return out[:N]


# speedup vs baseline: 11.2911x; 1.0804x over previous
import functools

import jax
import jax.numpy as jnp
from jax import lax
from jax.experimental import pallas as pl
from jax.experimental.pallas import tpu as pltpu
from jax.experimental.pallas import tpu_sc as plsc

N = 10000
D = 128
E = 320000
N_PAD = 10240
NW = 32
EPW = 10240
E_PAD = NW * EPW
CHUNK = 128
ROWS_PER_SC = N_PAD // 16

_mesh = plsc.VectorSubcoreMesh(core_axis_name="c", subcore_axis_name="s")
_sc_params = pltpu.CompilerParams(needs_layout_passes=False)


@functools.partial(
    pl.kernel,
    out_type=jax.ShapeDtypeStruct((2, N_PAD), jnp.float32),
    mesh=_mesh,
    compiler_params=_sc_params,
    scratch_types=[
        pltpu.VMEM((CHUNK,), jnp.int32),
        pltpu.VMEM((N_PAD,), jnp.float32),
        pltpu.VMEM((ROWS_PER_SC,), jnp.float32),
        pltpu.VMEM((ROWS_PER_SC,), jnp.float32),
        pltpu.VMEM_SHARED((16, N_PAD), jnp.float32),
    ],
)
def _deg_kernel(dst_hbm, out_hbm, idx_v, hist_v, col_v, buf_v, slab):
    cid = lax.axis_index("c")
    sid = lax.axis_index("s")
    wid = sid * 2 + cid
    zero16 = jnp.zeros((16,), jnp.float32)
    ones16 = jnp.ones((16,), jnp.float32)

    def zero_hist(i, carry):
        hist_v[pl.ds(i * 16, 16)] = zero16
        return carry

    lax.fori_loop(0, N_PAD // 16, zero_hist, 0)

    def chunk_body(ci, carry):
        base = wid * EPW + ci * CHUNK
        pltpu.sync_copy(dst_hbm.at[pl.ds(base, CHUNK)], idx_v)

        def inner(j, c2):
            idx = idx_v[pl.ds(j * 16, 16)]
            plsc.addupdate_scatter(hist_v, [idx], ones16)
            return c2

        lax.fori_loop(0, CHUNK // 16, inner, 0)
        return carry

    lax.fori_loop(0, EPW // CHUNK, chunk_body, 0)

    pltpu.sync_copy(hist_v, slab.at[sid])
    plsc.subcore_barrier()

    col0 = sid * ROWS_PER_SC

    def zero_col(i, carry):
        col_v[pl.ds(i * 16, 16)] = zero16
        return carry

    lax.fori_loop(0, ROWS_PER_SC // 16, zero_col, 0)

    def row_body(r, carry):
        pltpu.sync_copy(slab.at[r, pl.ds(col0, ROWS_PER_SC)], buf_v)

        def addv(i, c2):
            col_v[pl.ds(i * 16, 16)] = (
                col_v[pl.ds(i * 16, 16)] + buf_v[pl.ds(i * 16, 16)]
            )
            return c2

        lax.fori_loop(0, ROWS_PER_SC // 16, addv, 0)
        return carry

    lax.fori_loop(0, 16, row_body, 0)
    pltpu.sync_copy(col_v, out_hbm.at[cid, pl.ds(col0, ROWS_PER_SC)])


NCHUNK_TOT = E_PAD // CHUNK
NBUF = 2
C0 = 136
C1 = 24
assert 16 * (C0 + C1) == NCHUNK_TOT
assert C0 % (2 * NBUF) == 0 and C1 % (2 * NBUF) == 0


@functools.partial(
    pl.kernel,
    out_type=jax.ShapeDtypeStruct((2, N_PAD, D), jnp.float32),
    mesh=_mesh,
    compiler_params=_sc_params,
    scratch_types=(
        [pltpu.VMEM((2, NBUF, CHUNK), jnp.int32),
         pltpu.VMEM((2, NBUF, CHUNK), jnp.int32)]
        + [pltpu.VMEM((CHUNK, D), jnp.float32) for _ in range(NBUF)]
        + [pltpu.VMEM_SHARED((N_PAD, D), jnp.float32)]
        + [pltpu.SemaphoreType.DMA for _ in range(2 + 2 * NBUF)]
    ),
)
def _agg_kernel(h_hbm, src_hbm, dst_hbm, out_hbm, sidx_v, didx_v, *rest):
    rows = rest[:NBUF]
    acc_sh = rest[NBUF]
    isem = rest[NBUF + 1:NBUF + 3]
    gsem = rest[NBUF + 3:NBUF + 3 + NBUF]
    ssem = rest[NBUF + 3 + NBUF:]
    cid = lax.axis_index("c")
    sid = lax.axis_index("s")
    r0 = sid * ROWS_PER_SC
    row0 = jnp.where(cid == 0, sid * C0, 16 * C0 + sid * C1)
    ngroup = jnp.where(cid == 0, C0 // NBUF, C1 // NBUF)

    @pl.when(cid == 0)
    def _():
        pltpu.sync_copy(h_hbm.at[pl.ds(r0, ROWS_PER_SC)],
                        acc_sh.at[pl.ds(r0, ROWS_PER_SC)])

    @pl.when(cid == 1)
    def _():
        zero16 = jnp.zeros((16,), jnp.float32)

        def zrow(i, carry):
            for j in range(D // 16):
                rows[0][i, pl.ds(j * 16, 16)] = zero16
            return carry

        lax.fori_loop(0, CHUNK, zrow, 0)
        for k in range(ROWS_PER_SC // CHUNK):
            pltpu.sync_copy(rows[0], acc_sh.at[pl.ds(r0 + k * CHUNK, CHUNK)])

    plsc.subcore_barrier()

    def load_idx(g, p):
        blk = pl.ds(row0 + g * NBUF, NBUF)
        pltpu.async_copy(src_hbm.at[blk], sidx_v.at[p], isem[p])
        pltpu.async_copy(dst_hbm.at[blk], didx_v.at[p], isem[p])

    def wait_idx(p):
        pltpu.make_async_copy(src_hbm.at[pl.ds(0, NBUF)], sidx_v.at[p],
                              isem[p]).wait()
        pltpu.make_async_copy(dst_hbm.at[pl.ds(0, NBUF)], didx_v.at[p],
                              isem[p]).wait()

    load_idx(0, 0)
    load_idx(1, 1)
    wait_idx(0)
    for b in range(NBUF):
        pltpu.async_copy(h_hbm.at[sidx_v.at[0, b]], rows[b], gsem[b])

    def half_body(g, p):
        pnext = 1 - p
        for b in range(NBUF):
            pltpu.make_async_copy(h_hbm.at[sidx_v.at[p, b]], rows[b],
                                  gsem[b]).wait()
            pltpu.async_copy(rows[b], acc_sh.at[didx_v.at[p, b]], ssem[b],
                             add=True)
        @pl.when(g + 1 < ngroup)
        def _():
            wait_idx(pnext)
            for b in range(NBUF):
                pltpu.make_async_copy(rows[b], acc_sh.at[didx_v.at[p, b]],
                                      ssem[b]).wait()
                pltpu.async_copy(h_hbm.at[sidx_v.at[pnext, b]], rows[b],
                                 gsem[b])

        @pl.when(g + 2 < ngroup)
        def _():
            load_idx(g + 2, p)

    def group_body(q, carry):
        half_body(2 * q, 0)
        half_body(2 * q + 1, 1)
        return carry

    lax.fori_loop(0, ngroup // 2, group_body, 0)
    for b in range(NBUF):
        pltpu.make_async_copy(rows[b], acc_sh.at[didx_v.at[0, b]],
                              ssem[b]).wait()

    plsc.subcore_barrier()
    pltpu.sync_copy(acc_sh.at[pl.ds(r0, ROWS_PER_SC)],
                    out_hbm.at[cid, pl.ds(r0, ROWS_PER_SC)])


_BLK = 1024
_GRID = N_PAD // _BLK


def _h_body(p_ref, x_ref, w_ref, o_ref):
    deg = 1.0 + p_ref[0] + p_ref[1]
    dinv = lax.rsqrt(deg)
    h = lax.dot_general(x_ref[...], w_ref[...], (((1,), (1,)), ((), ())),
                        preferred_element_type=jnp.float32,
                        precision=lax.Precision.HIGHEST)
    o_ref[...] = dinv * h


def _first_layer(p, x_pad, W1):
    return pl.pallas_call(
        _h_body,
        grid=(_GRID,),
        in_specs=[
            pl.BlockSpec((2, _BLK, 1), lambda i: (0, i, 0)),
            pl.BlockSpec((_BLK, D), lambda i: (i, 0)),
            pl.BlockSpec((D, D), lambda i: (0, 0)),
        ],
        out_specs=pl.BlockSpec((_BLK, D), lambda i: (i, 0)),
        out_shape=jax.ShapeDtypeStruct((N_PAD, D), jnp.float32),
    )(p, x_pad, W1)


def _mid_body(p_ref, acc_ref, b_ref, w_ref, o_ref):
    deg = 1.0 + p_ref[0] + p_ref[1]
    dinv = lax.rsqrt(deg)
    agg = acc_ref[0] + acc_ref[1]
    x2 = dinv * agg + b_ref[...]
    h2 = lax.dot_general(x2, w_ref[...], (((1,), (1,)), ((), ())),
                         preferred_element_type=jnp.float32,
                         precision=lax.Precision.HIGHEST)
    o_ref[...] = dinv * h2


def _mid_layer(p, acc, b1, W2):
    return pl.pallas_call(
        _mid_body,
        grid=(_GRID,),
        in_specs=[
            pl.BlockSpec((2, _BLK, 1), lambda i: (0, i, 0)),
            pl.BlockSpec((2, _BLK, D), lambda i: (0, i, 0)),
            pl.BlockSpec((1, D), lambda i: (0, 0)),
            pl.BlockSpec((D, D), lambda i: (0, 0)),
        ],
        out_specs=pl.BlockSpec((_BLK, D), lambda i: (i, 0)),
        out_shape=jax.ShapeDtypeStruct((N_PAD, D), jnp.float32),
    )(p, acc, b1, W2)


def _final_body(p_ref, acc_ref, b_ref, o_ref):
    deg = 1.0 + p_ref[0] + p_ref[1]
    dinv = lax.rsqrt(deg)
    agg = acc_ref[0] + acc_ref[1]
    o_ref[...] = dinv * agg + b_ref[...]


def _final_layer(p, acc, b2):
    return pl.pallas_call(
        _final_body,
        grid=(_GRID,),
        in_specs=[
            pl.BlockSpec((2, _BLK, 1), lambda i: (0, i, 0)),
            pl.BlockSpec((2, _BLK, D), lambda i: (0, i, 0)),
            pl.BlockSpec((1, D), lambda i: (0, 0)),
        ],
        out_specs=pl.BlockSpec((_BLK, D), lambda i: (i, 0)),
        out_shape=jax.ShapeDtypeStruct((N_PAD, D), jnp.float32),
    )(p, acc, b2)


@jax.jit
def kernel(x, edge_index, W1, b1, W2, b2):
    src = edge_index[0].astype(jnp.int32)
    dst = edge_index[1].astype(jnp.int32)
    pad = jnp.full((E_PAD - E,), N, jnp.int32)
    src_pad = jnp.concatenate([src, pad])
    dst_pad = jnp.concatenate([dst, pad])
    src_m = src_pad.reshape(NCHUNK_TOT, CHUNK)
    dst_m = dst_pad.reshape(NCHUNK_TOT, CHUNK)
    x_pad = jnp.pad(x, ((0, N_PAD - N), (0, 0)))
    b1r = b1.reshape(1, D)
    b2r = b2.reshape(1, D)

    p = _deg_kernel(dst_pad)
    pcol = p.reshape(2, N_PAD, 1)

    h1 = _first_layer(pcol, x_pad, W1)
    acc1 = _agg_kernel(h1, src_m, dst_m)
    h2 = _mid_layer(pcol, acc1, b1r, W2)
    acc2 = _agg_kernel(h2, src_m, dst_m)
    out = _final_layer(pcol, acc2, b2r)
    return out[:N]


# speedup vs baseline: 11.4421x; 1.0134x over previous
import functools

import jax
import jax.numpy as jnp
from jax import lax
from jax.experimental import pallas as pl
from jax.experimental.pallas import tpu as pltpu
from jax.experimental.pallas import tpu_sc as plsc

N = 10000
D = 128
E = 320000
N_PAD = 10240
NW = 32
EPW = 10240
E_PAD = NW * EPW
CHUNK = 128
ROWS_PER_SC = N_PAD // 16

_mesh = plsc.VectorSubcoreMesh(core_axis_name="c", subcore_axis_name="s")
_sc_params = pltpu.CompilerParams(needs_layout_passes=False)


@functools.partial(
    pl.kernel,
    out_type=jax.ShapeDtypeStruct((2, N_PAD), jnp.float32),
    mesh=_mesh,
    compiler_params=_sc_params,
    scratch_types=[
        pltpu.VMEM((CHUNK,), jnp.int32),
        pltpu.VMEM((N_PAD,), jnp.float32),
        pltpu.VMEM((ROWS_PER_SC,), jnp.float32),
        pltpu.VMEM((ROWS_PER_SC,), jnp.float32),
        pltpu.VMEM_SHARED((16, N_PAD), jnp.float32),
    ],
)
def _deg_kernel(dst_hbm, out_hbm, idx_v, hist_v, col_v, buf_v, slab):
    cid = lax.axis_index("c")
    sid = lax.axis_index("s")
    wid = sid * 2 + cid
    zero16 = jnp.zeros((16,), jnp.float32)
    ones16 = jnp.ones((16,), jnp.float32)

    def zero_hist(i, carry):
        hist_v[pl.ds(i * 16, 16)] = zero16
        return carry

    lax.fori_loop(0, N_PAD // 16, zero_hist, 0)

    def chunk_body(ci, carry):
        base = wid * EPW + ci * CHUNK
        pltpu.sync_copy(dst_hbm.at[pl.ds(base, CHUNK)], idx_v)

        def inner(j, c2):
            idx = idx_v[pl.ds(j * 16, 16)]
            plsc.addupdate_scatter(hist_v, [idx], ones16)
            return c2

        lax.fori_loop(0, CHUNK // 16, inner, 0)
        return carry

    lax.fori_loop(0, EPW // CHUNK, chunk_body, 0)

    pltpu.sync_copy(hist_v, slab.at[sid])
    plsc.subcore_barrier()

    col0 = sid * ROWS_PER_SC

    def zero_col(i, carry):
        col_v[pl.ds(i * 16, 16)] = zero16
        return carry

    lax.fori_loop(0, ROWS_PER_SC // 16, zero_col, 0)

    def row_body(r, carry):
        pltpu.sync_copy(slab.at[r, pl.ds(col0, ROWS_PER_SC)], buf_v)

        def addv(i, c2):
            col_v[pl.ds(i * 16, 16)] = (
                col_v[pl.ds(i * 16, 16)] + buf_v[pl.ds(i * 16, 16)]
            )
            return c2

        lax.fori_loop(0, ROWS_PER_SC // 16, addv, 0)
        return carry

    lax.fori_loop(0, 16, row_body, 0)
    pltpu.sync_copy(col_v, out_hbm.at[cid, pl.ds(col0, ROWS_PER_SC)])


NCHUNK_TOT = E_PAD // CHUNK
NBUF = 2
C0 = 152
C1 = 8
assert 16 * (C0 + C1) == NCHUNK_TOT
assert C0 % (2 * NBUF) == 0 and C1 % (2 * NBUF) == 0


@functools.partial(
    pl.kernel,
    out_type=jax.ShapeDtypeStruct((2, N_PAD, D), jnp.float32),
    mesh=_mesh,
    compiler_params=_sc_params,
    scratch_types=(
        [pltpu.VMEM((2, NBUF, CHUNK), jnp.int32),
         pltpu.VMEM((2, NBUF, CHUNK), jnp.int32)]
        + [pltpu.VMEM((CHUNK, D), jnp.float32) for _ in range(NBUF)]
        + [pltpu.VMEM_SHARED((N_PAD, D), jnp.float32)]
        + [pltpu.SemaphoreType.DMA for _ in range(2 + 2 * NBUF)]
    ),
)
def _agg_kernel(h_hbm, src_hbm, dst_hbm, out_hbm, sidx_v, didx_v, *rest):
    rows = rest[:NBUF]
    acc_sh = rest[NBUF]
    isem = rest[NBUF + 1:NBUF + 3]
    gsem = rest[NBUF + 3:NBUF + 3 + NBUF]
    ssem = rest[NBUF + 3 + NBUF:]
    cid = lax.axis_index("c")
    sid = lax.axis_index("s")
    r0 = sid * ROWS_PER_SC
    row0 = jnp.where(cid == 0, sid * C0, 16 * C0 + sid * C1)
    ngroup = jnp.where(cid == 0, C0 // NBUF, C1 // NBUF)

    @pl.when(cid == 0)
    def _():
        pltpu.sync_copy(h_hbm.at[pl.ds(r0, ROWS_PER_SC)],
                        acc_sh.at[pl.ds(r0, ROWS_PER_SC)])

    @pl.when(cid == 1)
    def _():
        zero16 = jnp.zeros((16,), jnp.float32)

        def zrow(i, carry):
            for j in range(D // 16):
                rows[0][i, pl.ds(j * 16, 16)] = zero16
            return carry

        lax.fori_loop(0, CHUNK, zrow, 0)
        for k in range(ROWS_PER_SC // CHUNK):
            pltpu.sync_copy(rows[0], acc_sh.at[pl.ds(r0 + k * CHUNK, CHUNK)])

    plsc.subcore_barrier()

    def load_idx(g, p):
        blk = pl.ds(row0 + g * NBUF, NBUF)
        pltpu.async_copy(src_hbm.at[blk], sidx_v.at[p], isem[p])
        pltpu.async_copy(dst_hbm.at[blk], didx_v.at[p], isem[p])

    def wait_idx(p):
        pltpu.make_async_copy(src_hbm.at[pl.ds(0, NBUF)], sidx_v.at[p],
                              isem[p]).wait()
        pltpu.make_async_copy(dst_hbm.at[pl.ds(0, NBUF)], didx_v.at[p],
                              isem[p]).wait()

    load_idx(0, 0)
    load_idx(1, 1)
    wait_idx(0)
    for b in range(NBUF):
        pltpu.async_copy(h_hbm.at[sidx_v.at[0, b]], rows[b], gsem[b])

    def half_body(g, p):
        pnext = 1 - p
        for b in range(NBUF):
            pltpu.make_async_copy(h_hbm.at[sidx_v.at[p, b]], rows[b],
                                  gsem[b]).wait()
            pltpu.async_copy(rows[b], acc_sh.at[didx_v.at[p, b]], ssem[b],
                             add=True)
        @pl.when(g + 1 < ngroup)
        def _():
            wait_idx(pnext)
            for b in range(NBUF):
                pltpu.make_async_copy(rows[b], acc_sh.at[didx_v.at[p, b]],
                                      ssem[b]).wait()
                pltpu.async_copy(h_hbm.at[sidx_v.at[pnext, b]], rows[b],
                                 gsem[b])

        @pl.when(g + 2 < ngroup)
        def _():
            load_idx(g + 2, p)

    def group_body(q, carry):
        half_body(2 * q, 0)
        half_body(2 * q + 1, 1)
        return carry

    lax.fori_loop(0, ngroup // 2, group_body, 0)
    for b in range(NBUF):
        pltpu.make_async_copy(rows[b], acc_sh.at[didx_v.at[0, b]],
                              ssem[b]).wait()

    plsc.subcore_barrier()
    pltpu.sync_copy(acc_sh.at[pl.ds(r0, ROWS_PER_SC)],
                    out_hbm.at[cid, pl.ds(r0, ROWS_PER_SC)])


_BLK = 1024
_GRID = N_PAD // _BLK


def _h_body(p_ref, x_ref, w_ref, o_ref):
    deg = 1.0 + p_ref[0] + p_ref[1]
    dinv = lax.rsqrt(deg)
    h = lax.dot_general(x_ref[...], w_ref[...], (((1,), (1,)), ((), ())),
                        preferred_element_type=jnp.float32,
                        precision=lax.Precision.HIGHEST)
    o_ref[...] = dinv * h


def _first_layer(p, x_pad, W1):
    return pl.pallas_call(
        _h_body,
        grid=(_GRID,),
        in_specs=[
            pl.BlockSpec((2, _BLK, 1), lambda i: (0, i, 0)),
            pl.BlockSpec((_BLK, D), lambda i: (i, 0)),
            pl.BlockSpec((D, D), lambda i: (0, 0)),
        ],
        out_specs=pl.BlockSpec((_BLK, D), lambda i: (i, 0)),
        out_shape=jax.ShapeDtypeStruct((N_PAD, D), jnp.float32),
    )(p, x_pad, W1)


def _mid_body(p_ref, acc_ref, b_ref, w_ref, o_ref):
    deg = 1.0 + p_ref[0] + p_ref[1]
    dinv = lax.rsqrt(deg)
    agg = acc_ref[0] + acc_ref[1]
    x2 = dinv * agg + b_ref[...]
    h2 = lax.dot_general(x2, w_ref[...], (((1,), (1,)), ((), ())),
                         preferred_element_type=jnp.float32,
                         precision=lax.Precision.HIGHEST)
    o_ref[...] = dinv * h2


def _mid_layer(p, acc, b1, W2):
    return pl.pallas_call(
        _mid_body,
        grid=(_GRID,),
        in_specs=[
            pl.BlockSpec((2, _BLK, 1), lambda i: (0, i, 0)),
            pl.BlockSpec((2, _BLK, D), lambda i: (0, i, 0)),
            pl.BlockSpec((1, D), lambda i: (0, 0)),
            pl.BlockSpec((D, D), lambda i: (0, 0)),
        ],
        out_specs=pl.BlockSpec((_BLK, D), lambda i: (i, 0)),
        out_shape=jax.ShapeDtypeStruct((N_PAD, D), jnp.float32),
    )(p, acc, b1, W2)


def _final_body(p_ref, acc_ref, b_ref, o_ref):
    deg = 1.0 + p_ref[0] + p_ref[1]
    dinv = lax.rsqrt(deg)
    agg = acc_ref[0] + acc_ref[1]
    o_ref[...] = dinv * agg + b_ref[...]


def _final_layer(p, acc, b2):
    return pl.pallas_call(
        _final_body,
        grid=(_GRID,),
        in_specs=[
            pl.BlockSpec((2, _BLK, 1), lambda i: (0, i, 0)),
            pl.BlockSpec((2, _BLK, D), lambda i: (0, i, 0)),
            pl.BlockSpec((1, D), lambda i: (0, 0)),
        ],
        out_specs=pl.BlockSpec((_BLK, D), lambda i: (i, 0)),
        out_shape=jax.ShapeDtypeStruct((N_PAD, D), jnp.float32),
    )(p, acc, b2)


@jax.jit
def kernel(x, edge_index, W1, b1, W2, b2):
    src = edge_index[0].astype(jnp.int32)
    dst = edge_index[1].astype(jnp.int32)
    pad = jnp.full((E_PAD - E,), N, jnp.int32)
    src_pad = jnp.concatenate([src, pad])
    dst_pad = jnp.concatenate([dst, pad])
    src_m = src_pad.reshape(NCHUNK_TOT, CHUNK)
    dst_m = dst_pad.reshape(NCHUNK_TOT, CHUNK)
    x_pad = jnp.pad(x, ((0, N_PAD - N), (0, 0)))
    b1r = b1.reshape(1, D)
    b2r = b2.reshape(1, D)

    p = _deg_kernel(dst_pad)
    pcol = p.reshape(2, N_PAD, 1)

    h1 = _first_layer(pcol, x_pad, W1)
    acc1 = _agg_kernel(h1, src_m, dst_m)
    h2 = _mid_layer(pcol, acc1, b1r, W2)
    acc2 = _agg_kernel(h2, src_m, dst_m)
    out = _final_layer(pcol, acc2, b2r)
    return out[:N]
